# Initial kernel scaffold; baseline (speedup 1.0000x reference)
import jax
import jax.numpy as jnp
from jax.experimental import pallas as pl


def kernel(H, A, det_W1, det_b1, det_W2, det_b2, res_W1, res_b1, res_W2, res_b2):
    raise NotImplementedError("write your pallas kernel here")


# speedup vs baseline: 2.9920x; 2.9920x over previous
import jax
import jax.numpy as jnp
from jax.experimental import pallas as pl
from jax.experimental.pallas import tpu as pltpu

_N = 256
_FD = 128
_ROWS = 32
_THRESHOLD = 0.5


def _elu(x):
    return jnp.where(x > 0, x, jnp.exp(x) - 1.0)


def _resolver_kernel(H_ref, A_ref, dW1a_ref, dW1b_ref, db1_ref, dw2_ref,
                     db2_ref, rW1h_ref, rW1n_ref, rw1c_ref, rb1_ref, rW2_ref,
                     rb2_ref, out_ref, mask_ref):
    H = H_ref[0]
    A = A_ref[0]

    P = jnp.dot(H, dW1a_ref[...], preferred_element_type=jnp.float32) + db1_ref[...]
    Q = jnp.dot(H, dW1b_ref[...], preferred_element_type=jnp.float32)
    w2 = dw2_ref[...]
    b2 = db2_ref[0, 0]

    cmax_parts = []
    for c in range(_N // _ROWS):
        Pc = P[c * _ROWS:(c + 1) * _ROWS]
        T = _elu(Pc[:, None, :] + Q[None, :, :])
        z = jnp.sum(T * w2[None, :, :], axis=2)
        s = jax.nn.sigmoid(z + b2)
        Ac = A[c * _ROWS:(c + 1) * _ROWS]
        contra = jnp.where(Ac > 0.1, s, 0.0)
        cmax_parts.append(jnp.max(contra, axis=1, keepdims=True))
    cmax = jnp.concatenate(cmax_parts, axis=0)

    nb = (A > 0).astype(jnp.float32)
    cnt = jnp.sum(nb, axis=1, keepdims=True)
    nfeat = jnp.dot(nb, H, preferred_element_type=jnp.float32) / jnp.maximum(cnt, 1.0)

    pre = (jnp.dot(H, rW1h_ref[...], preferred_element_type=jnp.float32)
           + jnp.dot(nfeat, rW1n_ref[...], preferred_element_type=jnp.float32)
           + cmax * rw1c_ref[...]
           + rb1_ref[...])
    resolved = jnp.dot(_elu(pre), rW2_ref[...], preferred_element_type=jnp.float32) + rb2_ref[...]

    apply = (cmax > _THRESHOLD) & (cnt > 0)
    out_ref[0] = jnp.where(apply, resolved, H)
    mask_ref[0] = apply.astype(jnp.float32)


def kernel(H, A, det_W1, det_b1, det_W2, det_b2, res_W1, res_b1, res_W2, res_b2):
    Bb = H.shape[0]
    dW1a = det_W1[:_FD]
    dW1b = det_W1[_FD:]
    rW1h = res_W1[:_FD]
    rW1n = res_W1[_FD:2 * _FD]
    rw1c = res_W1[2 * _FD:]
    db1 = det_b1[None, :]
    dw2 = det_W2.T
    db2 = det_b2[None, :]
    rb1 = res_b1[None, :]
    rb2 = res_b2[None, :]

    wspec = lambda shape: pl.BlockSpec(shape, lambda b: (0,) * len(shape))
    out, maskf = pl.pallas_call(
        _resolver_kernel,
        grid=(Bb,),
        in_specs=[
            pl.BlockSpec((1, _N, _FD), lambda b: (b, 0, 0)),
            pl.BlockSpec((1, _N, _N), lambda b: (b, 0, 0)),
            wspec((_FD, _FD)),
            wspec((_FD, _FD)),
            wspec((1, _FD)),
            wspec((1, _FD)),
            wspec((1, 1)),
            wspec((_FD, _FD)),
            wspec((_FD, _FD)),
            wspec((1, _FD)),
            wspec((1, _FD)),
            wspec((_FD, _FD)),
            wspec((1, _FD)),
        ],
        out_specs=[
            pl.BlockSpec((1, _N, _FD), lambda b: (b, 0, 0)),
            pl.BlockSpec((1, _N, 1), lambda b: (b, 0, 0)),
        ],
        out_shape=[
            jax.ShapeDtypeStruct((Bb, _N, _FD), jnp.float32),
            jax.ShapeDtypeStruct((Bb, _N, 1), jnp.float32),
        ],
        compiler_params=pltpu.CompilerParams(dimension_semantics=("parallel",)),
    )(H, A, dW1a, dW1b, db1, dw2, db2, rW1h, rW1n, rw1c, rb1, res_W2, rb2)
    return out, (maskf[..., 0] > 0.5)


# speedup vs baseline: 4.1494x; 1.3868x over previous
import jax
import jax.numpy as jnp
from jax.experimental import pallas as pl
from jax.experimental.pallas import tpu as pltpu

_N = 256
_FD = 128
_ROWS = 32
_THRESHOLD = 0.5


def _elu(x):
    return jnp.where(x > 0, x, jnp.exp(x) - 1.0)


def _resolver_kernel(H_ref, A_ref, dW1a_ref, dW1b_ref, db1_ref, dw2_ref,
                     db2_ref, rW1h_ref, rW1n_ref, rw1c_ref, rb1_ref, rW2_ref,
                     rb2_ref, out_ref, mask_ref):
    H = H_ref[0]
    A = A_ref[0]

    P = jnp.dot(H, dW1a_ref[...], preferred_element_type=jnp.float32) + db1_ref[...]
    Q = jnp.dot(H, dW1b_ref[...], preferred_element_type=jnp.float32)
    w2 = dw2_ref[...]
    b2 = db2_ref[0, 0]

    EP = jnp.exp(P)
    EQ = jnp.exp(Q)
    w2sum = jnp.sum(w2)

    cmax_parts = []
    for c in range(_N // _ROWS):
        Pc = P[c * _ROWS:(c + 1) * _ROWS]
        EPc = EP[c * _ROWS:(c + 1) * _ROWS]
        t = Pc[:, None, :] + Q[None, :, :]
        u = jnp.maximum(t, 0.0) + jnp.minimum(EPc[:, None, :] * EQ[None, :, :], 1.0)
        z = jnp.sum(u * w2[None, :, :], axis=2) - w2sum
        Ac = A[c * _ROWS:(c + 1) * _ROWS]
        m = jnp.where(Ac > 0.1, z, -1e30)
        zmax = jnp.max(m, axis=1, keepdims=True)
        cmax_parts.append(
            jnp.where(zmax > -1e29, jax.nn.sigmoid(zmax + b2), 0.0))
    cmax = jnp.concatenate(cmax_parts, axis=0)

    nb = (A > 0).astype(jnp.float32)
    cnt = jnp.sum(nb, axis=1, keepdims=True)
    nfeat = jnp.dot(nb, H, preferred_element_type=jnp.float32) / jnp.maximum(cnt, 1.0)

    pre = (jnp.dot(H, rW1h_ref[...], preferred_element_type=jnp.float32)
           + jnp.dot(nfeat, rW1n_ref[...], preferred_element_type=jnp.float32)
           + cmax * rw1c_ref[...]
           + rb1_ref[...])
    resolved = jnp.dot(_elu(pre), rW2_ref[...], preferred_element_type=jnp.float32) + rb2_ref[...]

    apply = (cmax > _THRESHOLD) & (cnt > 0)
    out_ref[0] = jnp.where(apply, resolved, H)
    mask_ref[0] = apply.astype(jnp.float32)


def kernel(H, A, det_W1, det_b1, det_W2, det_b2, res_W1, res_b1, res_W2, res_b2):
    Bb = H.shape[0]
    dW1a = det_W1[:_FD]
    dW1b = det_W1[_FD:]
    rW1h = res_W1[:_FD]
    rW1n = res_W1[_FD:2 * _FD]
    rw1c = res_W1[2 * _FD:]
    db1 = det_b1[None, :]
    dw2 = det_W2.T
    db2 = det_b2[None, :]
    rb1 = res_b1[None, :]
    rb2 = res_b2[None, :]

    wspec = lambda shape: pl.BlockSpec(shape, lambda b: (0,) * len(shape))
    out, maskf = pl.pallas_call(
        _resolver_kernel,
        grid=(Bb,),
        in_specs=[
            pl.BlockSpec((1, _N, _FD), lambda b: (b, 0, 0)),
            pl.BlockSpec((1, _N, _N), lambda b: (b, 0, 0)),
            wspec((_FD, _FD)),
            wspec((_FD, _FD)),
            wspec((1, _FD)),
            wspec((1, _FD)),
            wspec((1, 1)),
            wspec((_FD, _FD)),
            wspec((_FD, _FD)),
            wspec((1, _FD)),
            wspec((1, _FD)),
            wspec((_FD, _FD)),
            wspec((1, _FD)),
        ],
        out_specs=[
            pl.BlockSpec((1, _N, _FD), lambda b: (b, 0, 0)),
            pl.BlockSpec((1, _N, 1), lambda b: (b, 0, 0)),
        ],
        out_shape=[
            jax.ShapeDtypeStruct((Bb, _N, _FD), jnp.float32),
            jax.ShapeDtypeStruct((Bb, _N, 1), jnp.float32),
        ],
        compiler_params=pltpu.CompilerParams(dimension_semantics=("parallel",)),
    )(H, A, dW1a, dW1b, db1, dw2, db2, rW1h, rW1n, rw1c, rb1, res_W2, rb2)
    return out, (maskf[..., 0] > 0.5)


# speedup vs baseline: 4.4559x; 1.0739x over previous
import jax
import jax.numpy as jnp
from jax.experimental import pallas as pl
from jax.experimental.pallas import tpu as pltpu

_N = 256
_FD = 128
_ROWS = 32
_THRESHOLD = 0.5


def _elu(x):
    return jnp.where(x > 0, x, jnp.exp(x) - 1.0)


def _resolver_kernel(H_ref, A_ref, dW1a_ref, dW1b_ref, db1_ref, dw2_ref,
                     db2_ref, rW1h_ref, rW1n_ref, rw1c_ref, rb1_ref, rW2_ref,
                     rb2_ref, out_ref, mask_ref):
    H = H_ref[0]
    A = A_ref[0]

    P = jnp.dot(H, dW1a_ref[...], preferred_element_type=jnp.float32) + db1_ref[...]
    Q = jnp.dot(H, dW1b_ref[...], preferred_element_type=jnp.float32)
    w2 = dw2_ref[...]
    b2 = db2_ref[0, 0]

    EP = jnp.exp(P)
    EQ = jnp.exp(Q)
    w2sum = jnp.sum(w2)

    cmax_parts = []
    for c in range(_N // _ROWS):
        Pc = P[c * _ROWS:(c + 1) * _ROWS]
        EPc = EP[c * _ROWS:(c + 1) * _ROWS]
        t = Pc[:, None, :] + Q[None, :, :]
        u = jnp.maximum(t, 0.0) + jnp.minimum(EPc[:, None, :] * EQ[None, :, :], 1.0)
        z = jnp.sum(u * w2[None, :, :], axis=2)
        Ac = A[c * _ROWS:(c + 1) * _ROWS]
        m = jnp.where(Ac > 0.1, z, -1e30)
        zmax = jnp.max(m, axis=1, keepdims=True)
        cmax_parts.append(
            jnp.where(zmax > -1e29, jax.nn.sigmoid(zmax - w2sum + b2), 0.0))
    cmax = jnp.concatenate(cmax_parts, axis=0)

    nb = (A > 0).astype(jnp.float32)
    cnt = jnp.sum(nb, axis=1, keepdims=True)
    nfeat = jnp.dot(nb, H, preferred_element_type=jnp.float32) / jnp.maximum(cnt, 1.0)

    pre = (jnp.dot(H, rW1h_ref[...], preferred_element_type=jnp.float32)
           + jnp.dot(nfeat, rW1n_ref[...], preferred_element_type=jnp.float32)
           + cmax * rw1c_ref[...]
           + rb1_ref[...])
    resolved = jnp.dot(_elu(pre), rW2_ref[...], preferred_element_type=jnp.float32) + rb2_ref[...]

    apply = (cmax > _THRESHOLD) & (cnt > 0)
    out_ref[0] = jnp.where(apply, resolved, H)
    mask_ref[0] = apply.astype(jnp.float32)


def kernel(H, A, det_W1, det_b1, det_W2, det_b2, res_W1, res_b1, res_W2, res_b2):
    Bb = H.shape[0]
    dW1a = det_W1[:_FD]
    dW1b = det_W1[_FD:]
    rW1h = res_W1[:_FD]
    rW1n = res_W1[_FD:2 * _FD]
    rw1c = res_W1[2 * _FD:]
    db1 = det_b1[None, :]
    dw2 = det_W2.T
    db2 = det_b2[None, :]
    rb1 = res_b1[None, :]
    rb2 = res_b2[None, :]

    wspec = lambda shape: pl.BlockSpec(shape, lambda b: (0,) * len(shape))
    out, maskf = pl.pallas_call(
        _resolver_kernel,
        grid=(Bb,),
        in_specs=[
            pl.BlockSpec((1, _N, _FD), lambda b: (b, 0, 0)),
            pl.BlockSpec((1, _N, _N), lambda b: (b, 0, 0)),
            wspec((_FD, _FD)),
            wspec((_FD, _FD)),
            wspec((1, _FD)),
            wspec((1, _FD)),
            wspec((1, 1)),
            wspec((_FD, _FD)),
            wspec((_FD, _FD)),
            wspec((1, _FD)),
            wspec((1, _FD)),
            wspec((_FD, _FD)),
            wspec((1, _FD)),
        ],
        out_specs=[
            pl.BlockSpec((1, _N, _FD), lambda b: (b, 0, 0)),
            pl.BlockSpec((1, _N, 1), lambda b: (b, 0, 0)),
        ],
        out_shape=[
            jax.ShapeDtypeStruct((Bb, _N, _FD), jnp.float32),
            jax.ShapeDtypeStruct((Bb, _N, 1), jnp.float32),
        ],
        compiler_params=pltpu.CompilerParams(dimension_semantics=("parallel",)),
    )(H, A, dW1a, dW1b, db1, dw2, db2, rW1h, rW1n, rw1c, rb1, res_W2, rb2)
    return out, (maskf[..., 0] > 0.5)


# speedup vs baseline: 4.7265x; 1.0607x over previous
import jax
import jax.numpy as jnp
from jax.experimental import pallas as pl
from jax.experimental.pallas import tpu as pltpu

_N = 256
_FD = 128
_ROWS = 32
_THRESHOLD = 0.5


def _elu(x):
    return jnp.where(x > 0, x, jnp.exp(x) - 1.0)


def _resolver_kernel(H_ref, A_ref, dW1a_ref, dW1b_ref, db1_ref, dw2_ref,
                     db2_ref, rW1h_ref, rW1n_ref, rw1c_ref, rb1_ref, rW2_ref,
                     rb2_ref, out_ref, mask_ref):
    H = H_ref[0]
    A = A_ref[0]

    P = jnp.dot(H, dW1a_ref[...], preferred_element_type=jnp.float32) + db1_ref[...]
    Q = jnp.dot(H, dW1b_ref[...], preferred_element_type=jnp.float32)
    w2 = dw2_ref[...]
    b2 = db2_ref[0, 0]

    EP = jnp.exp(P)
    EQ = jnp.exp(Q)
    Q1 = Q + 1.0
    w2sum = jnp.sum(w2)

    cmax_parts = []
    for c in range(_N // _ROWS):
        Pc = P[c * _ROWS:(c + 1) * _ROWS]
        EPc = EP[c * _ROWS:(c + 1) * _ROWS]
        t1 = Pc[:, None, :] + Q1[None, :, :]
        u = jnp.maximum(t1, jnp.minimum(EPc[:, None, :] * EQ[None, :, :], 1.0))
        z = jnp.sum(u * w2[None, :, :], axis=2)
        Ac = A[c * _ROWS:(c + 1) * _ROWS]
        m = jnp.where(Ac > 0.1, z, -1e30)
        zmax = jnp.max(m, axis=1, keepdims=True)
        cmax_parts.append(
            jnp.where(zmax > -1e29, jax.nn.sigmoid(zmax - w2sum + b2), 0.0))
    cmax = jnp.concatenate(cmax_parts, axis=0)

    nb = (A > 0).astype(jnp.float32)
    cnt = jnp.sum(nb, axis=1, keepdims=True)
    nfeat = jnp.dot(nb, H, preferred_element_type=jnp.float32) / jnp.maximum(cnt, 1.0)

    pre = (jnp.dot(H, rW1h_ref[...], preferred_element_type=jnp.float32)
           + jnp.dot(nfeat, rW1n_ref[...], preferred_element_type=jnp.float32)
           + cmax * rw1c_ref[...]
           + rb1_ref[...])
    resolved = jnp.dot(_elu(pre), rW2_ref[...], preferred_element_type=jnp.float32) + rb2_ref[...]

    apply = (cmax > _THRESHOLD) & (cnt > 0)
    out_ref[0] = jnp.where(apply, resolved, H)
    mask_ref[0] = apply.astype(jnp.float32)


def kernel(H, A, det_W1, det_b1, det_W2, det_b2, res_W1, res_b1, res_W2, res_b2):
    Bb = H.shape[0]
    db1 = det_b1[None, :]
    dw2 = det_W2.T
    db2 = det_b2[None, :]
    rw1c = res_W1[2 * _FD:]
    rb1 = res_b1[None, :]
    rb2 = res_b2[None, :]

    wspec = lambda shape: pl.BlockSpec(shape, lambda b: (0,) * len(shape))
    out, maskf = pl.pallas_call(
        _resolver_kernel,
        grid=(Bb,),
        in_specs=[
            pl.BlockSpec((1, _N, _FD), lambda b: (b, 0, 0)),
            pl.BlockSpec((1, _N, _N), lambda b: (b, 0, 0)),
            pl.BlockSpec((_FD, _FD), lambda b: (0, 0)),
            pl.BlockSpec((_FD, _FD), lambda b: (1, 0)),
            wspec((1, _FD)),
            wspec((1, _FD)),
            wspec((1, 1)),
            pl.BlockSpec((_FD, _FD), lambda b: (0, 0)),
            pl.BlockSpec((_FD, _FD), lambda b: (1, 0)),
            wspec((1, _FD)),
            wspec((1, _FD)),
            wspec((_FD, _FD)),
            wspec((1, _FD)),
        ],
        out_specs=[
            pl.BlockSpec((1, _N, _FD), lambda b: (b, 0, 0)),
            pl.BlockSpec((1, _N, 1), lambda b: (b, 0, 0)),
        ],
        out_shape=[
            jax.ShapeDtypeStruct((Bb, _N, _FD), jnp.float32),
            jax.ShapeDtypeStruct((Bb, _N, 1), jnp.float32),
        ],
        compiler_params=pltpu.CompilerParams(dimension_semantics=("parallel",)),
    )(H, A, det_W1, det_W1, db1, dw2, db2, res_W1, res_W1, rw1c, rb1, res_W2, rb2)
    return out, (maskf[..., 0] > 0.5)


# speedup vs baseline: 4.8866x; 1.0339x over previous
import jax
import jax.numpy as jnp
from jax.experimental import pallas as pl
from jax.experimental.pallas import tpu as pltpu

_N = 256
_FD = 128
_ROWS = 128
_THRESHOLD = 0.5


def _elu(x):
    return jnp.where(x > 0, x, jnp.exp(x) - 1.0)


def _resolver_kernel(H_ref, A_ref, dW1a_ref, dW1b_ref, db1_ref, dw2_ref,
                     db2_ref, rW1h_ref, rW1n_ref, rw1c_ref, rb1_ref, rW2_ref,
                     rb2_ref, out_ref, mask_ref):
    H = H_ref[0]
    A = A_ref[0]

    P = jnp.dot(H, dW1a_ref[...], preferred_element_type=jnp.float32) + db1_ref[...]
    Q = jnp.dot(H, dW1b_ref[...], preferred_element_type=jnp.float32)
    w2 = dw2_ref[...]
    b2 = db2_ref[0, 0]

    EP = jnp.exp(P)
    EQ = jnp.exp(Q)
    Q1 = Q + 1.0
    w2sum = jnp.sum(w2)

    cmax_parts = []
    for c in range(_N // _ROWS):
        Pc = P[c * _ROWS:(c + 1) * _ROWS]
        EPc = EP[c * _ROWS:(c + 1) * _ROWS]
        t1 = Pc[:, None, :] + Q1[None, :, :]
        u = jnp.maximum(t1, jnp.minimum(EPc[:, None, :] * EQ[None, :, :], 1.0))
        z = jnp.sum(u * w2[None, :, :], axis=2)
        Ac = A[c * _ROWS:(c + 1) * _ROWS]
        m = jnp.where(Ac > 0.1, z, -1e30)
        zmax = jnp.max(m, axis=1, keepdims=True)
        cmax_parts.append(
            jnp.where(zmax > -1e29, jax.nn.sigmoid(zmax - w2sum + b2), 0.0))
    cmax = jnp.concatenate(cmax_parts, axis=0)

    nb = (A > 0).astype(jnp.float32)
    cnt = jnp.sum(nb, axis=1, keepdims=True)
    nfeat = jnp.dot(nb, H, preferred_element_type=jnp.float32) / jnp.maximum(cnt, 1.0)

    pre = (jnp.dot(H, rW1h_ref[...], preferred_element_type=jnp.float32)
           + jnp.dot(nfeat, rW1n_ref[...], preferred_element_type=jnp.float32)
           + cmax * rw1c_ref[...]
           + rb1_ref[...])
    resolved = jnp.dot(_elu(pre), rW2_ref[...], preferred_element_type=jnp.float32) + rb2_ref[...]

    apply = (cmax > _THRESHOLD) & (cnt > 0)
    out_ref[0] = jnp.where(apply, resolved, H)
    mask_ref[0] = apply.astype(jnp.float32)


def kernel(H, A, det_W1, det_b1, det_W2, det_b2, res_W1, res_b1, res_W2, res_b2):
    Bb = H.shape[0]
    db1 = det_b1[None, :]
    dw2 = det_W2.T
    db2 = det_b2[None, :]
    rw1c = res_W1[2 * _FD:]
    rb1 = res_b1[None, :]
    rb2 = res_b2[None, :]

    wspec = lambda shape: pl.BlockSpec(shape, lambda b: (0,) * len(shape))
    out, maskf = pl.pallas_call(
        _resolver_kernel,
        grid=(Bb,),
        in_specs=[
            pl.BlockSpec((1, _N, _FD), lambda b: (b, 0, 0)),
            pl.BlockSpec((1, _N, _N), lambda b: (b, 0, 0)),
            pl.BlockSpec((_FD, _FD), lambda b: (0, 0)),
            pl.BlockSpec((_FD, _FD), lambda b: (1, 0)),
            wspec((1, _FD)),
            wspec((1, _FD)),
            wspec((1, 1)),
            pl.BlockSpec((_FD, _FD), lambda b: (0, 0)),
            pl.BlockSpec((_FD, _FD), lambda b: (1, 0)),
            wspec((1, _FD)),
            wspec((1, _FD)),
            wspec((_FD, _FD)),
            wspec((1, _FD)),
        ],
        out_specs=[
            pl.BlockSpec((1, _N, _FD), lambda b: (b, 0, 0)),
            pl.BlockSpec((1, _N, 1), lambda b: (b, 0, 0)),
        ],
        out_shape=[
            jax.ShapeDtypeStruct((Bb, _N, _FD), jnp.float32),
            jax.ShapeDtypeStruct((Bb, _N, 1), jnp.float32),
        ],
        compiler_params=pltpu.CompilerParams(dimension_semantics=("parallel",)),
    )(H, A, det_W1, det_W1, db1, dw2, db2, res_W1, res_W1, rw1c, rb1, res_W2, rb2)
    return out, (maskf[..., 0] > 0.5)
